# baseline (speedup 1.0000x reference)
import jax
import jax.numpy as jnp
from jax import lax
from jax.experimental import pallas as pl
from jax.experimental.pallas import tpu as pltpu
from jax.experimental.pallas import tpu_sc as plsc

D_MODEL = 64
SCALE = float(D_MODEL) ** 0.5
NUM_CORES = 2
NUM_SUBCORES = 16
NUM_WORKERS = NUM_CORES * NUM_SUBCORES
CHUNK = 512


def _emb_body(b_per_w, x_hbm, table_hbm, out_hbm, idx_v, rows_v, sem):
    wid = lax.axis_index("s") * NUM_CORES + lax.axis_index("c")
    base = wid * b_per_w
    n_chunks = b_per_w // CHUNK

    def chunk_body(c, carry):
        off = base + c * CHUNK
        pltpu.sync_copy(x_hbm.at[pl.ds(off, CHUNK)], idx_v)
        pltpu.async_copy(table_hbm.at[idx_v], rows_v, sem).wait()

        def scale_body(r, carry2):
            for j in range(D_MODEL // 16):
                sl = pl.ds(j * 16, 16)
                rows_v[r, sl] = rows_v[r, sl] * SCALE
            return carry2

        lax.fori_loop(0, CHUNK, scale_body, 0, unroll=4)
        pltpu.sync_copy(rows_v, out_hbm.at[pl.ds(off, CHUNK)])
        return carry

    lax.fori_loop(0, n_chunks, chunk_body, 0)


@jax.jit
def kernel(x, table):
    batch, seq = x.shape
    n_rows = batch * seq
    assert n_rows % (NUM_WORKERS * CHUNK) == 0
    b_per_w = n_rows // NUM_WORKERS
    xf = x.reshape(n_rows).astype(jnp.int32)

    mesh = plsc.VectorSubcoreMesh(
        core_axis_name="c",
        subcore_axis_name="s",
        num_cores=NUM_CORES,
        num_subcores=NUM_SUBCORES,
    )
    out = pl.kernel(
        lambda *refs: _emb_body(b_per_w, *refs),
        out_type=jax.ShapeDtypeStruct((n_rows, D_MODEL), jnp.float32),
        mesh=mesh,
        compiler_params=pltpu.CompilerParams(use_tc_tiling_on_sc=False),
        scratch_types=[
            pltpu.VMEM((CHUNK,), jnp.int32),
            pltpu.VMEM((CHUNK, D_MODEL), jnp.float32),
            pltpu.SemaphoreType.DMA,
        ],
    )(xf, table)
    return out.reshape(batch, seq, D_MODEL)


# speedup vs baseline: 1.0918x; 1.0918x over previous
import functools

import jax
import jax.numpy as jnp
from jax import lax
from jax.experimental import pallas as pl
from jax.experimental.pallas import tpu as pltpu
from jax.experimental.pallas import tpu_sc as plsc

D_MODEL = 64
SCALE = float(D_MODEL) ** 0.5
NUM_CORES = 2
NUM_SUBCORES = 16
NUM_WORKERS = NUM_CORES * NUM_SUBCORES
CHUNK = 400
NBUF = 4
AHEAD = 2


def _emb_body(b_per_w, x_hbm, table_hbm, out_hbm, i0, i1, i2, i3,
              r0, r1, r2, r3, g0, g1, g2, g3, o0, o1, o2, o3):
    idx = (i0, i1, i2, i3)
    rows = (r0, r1, r2, r3)
    gsem = (g0, g1, g2, g3)
    osem = (o0, o1, o2, o3)
    wid = lax.axis_index("s") * NUM_CORES + lax.axis_index("c")
    base = wid * b_per_w
    n_chunks = b_per_w // CHUNK

    def gather_copy(g, b):
        return pltpu.make_async_copy(table_hbm.at[idx[b]], rows[b], gsem[b])

    def load_idx(g, b):
        pltpu.sync_copy(x_hbm.at[pl.ds(base + g * CHUNK, CHUNK)], idx[b])

    def out_copy(g, b):
        return pltpu.make_async_copy(
            rows[b], out_hbm.at[pl.ds(base + g * CHUNK, CHUNK)], osem[b])

    def scale(b):
        rv = rows[b]

        @plsc.parallel_loop(0, CHUNK, unroll=8)
        def _(r):
            for j in range(D_MODEL // 16):
                sl = pl.ds(j * 16, 16)
                rv[r, sl] = rv[r, sl] * SCALE

    def unit(g, b, wait_prev_out, issue_next):
        if issue_next:
            bn = (b + AHEAD) % NBUF
            if wait_prev_out:
                out_copy(0, bn).wait()
            load_idx(g + AHEAD, bn)
            gather_copy(g + AHEAD, bn).start()
        gather_copy(g, b).wait()
        scale(b)
        out_copy(g, b).start()

    for g in range(AHEAD):
        load_idx(g, g % NBUF)
        gather_copy(g, g % NBUF).start()
    for g in range(NBUF):
        unit(g, g % NBUF, wait_prev_out=(g >= AHEAD), issue_next=True)

    def pbody(p, carry):
        g0_ = p * NBUF
        for b in range(NBUF):
            unit(g0_ + b, b, wait_prev_out=True, issue_next=True)
        return carry

    lax.fori_loop(1, n_chunks // NBUF - 1, pbody, 0)

    for g in range(n_chunks - NBUF, n_chunks):
        unit(g, g % NBUF, wait_prev_out=True,
             issue_next=(g + AHEAD < n_chunks))
    for b in range(NBUF):
        out_copy(0, b).wait()


@jax.jit
def kernel(x, table):
    batch, seq = x.shape
    n_rows = batch * seq
    assert n_rows % (NUM_WORKERS * CHUNK * NBUF) == 0
    b_per_w = n_rows // NUM_WORKERS
    xf = x.reshape(n_rows).astype(jnp.int32)

    mesh = plsc.VectorSubcoreMesh(
        core_axis_name="c",
        subcore_axis_name="s",
        num_cores=NUM_CORES,
        num_subcores=NUM_SUBCORES,
    )
    out = pl.kernel(
        lambda *refs: _emb_body(b_per_w, *refs),
        out_type=jax.ShapeDtypeStruct((n_rows, D_MODEL), jnp.float32),
        mesh=mesh,
        compiler_params=pltpu.CompilerParams(use_tc_tiling_on_sc=False),
        scratch_types=[pltpu.VMEM((CHUNK,), jnp.int32) for _ in range(NBUF)]
        + [pltpu.VMEM((CHUNK, D_MODEL), jnp.float32) for _ in range(NBUF)]
        + [pltpu.SemaphoreType.DMA for _ in range(2 * NBUF)],
    )(xf, table)
    return out.reshape(batch, seq, D_MODEL)
